# baseline (speedup 1.0000x reference)
import functools

import jax
import jax.numpy as jnp
import numpy as np
from jax import lax
from jax.experimental import pallas as pl
from jax.experimental.pallas import tpu as pltpu
from jax.experimental.pallas import tpu_sc as plsc

B = 128
N = 32768
K = 64
L = 16
CHUNK = 256
NCH = N // CHUNK
NG = NCH // L
NW = 32
RPW = B // NW
BIG = np.int32(2**30)
NEGINF = np.float32(-np.inf)

_mesh = plsc.VectorSubcoreMesh(core_axis_name="c", subcore_axis_name="s")


@functools.partial(
    pl.kernel,
    mesh=_mesh,
    compiler_params=pltpu.CompilerParams(needs_layout_passes=False),
    out_type=[
        jax.ShapeDtypeStruct((B, K), jnp.float32),
        jax.ShapeDtypeStruct((B, K), jnp.int32),
    ],
    scratch_types=[
        pltpu.VMEM((N,), jnp.float32),
        pltpu.VMEM((N,), jnp.float32),
        pltpu.VMEM((N,), jnp.float32),
        pltpu.VMEM((K,), jnp.float32),
        pltpu.VMEM((K,), jnp.float32),
        pltpu.VMEM((K,), jnp.int32),
        pltpu.VMEM((K,), jnp.int32),
        pltpu.SemaphoreType.DMA,
        pltpu.SemaphoreType.DMA,
    ],
)
def _topk_sc(x_hbm, vals_hbm, idx_hbm, bufa, bufb, bufc, va, vb, ia, ib,
             sem0, sem1):
    wid = lax.axis_index("s") * 2 + lax.axis_index("c")
    lane = lax.iota(jnp.int32, L)
    lane0 = lane == 0

    _gdn = lax.GatherDimensionNumbers(
        offset_dims=(), collapsed_slice_dims=(0,), start_index_map=(0,))

    def _shuffle(v, perm):
        return lax.gather(
            v, perm[:, None], dimension_numbers=_gdn, slice_sizes=(1,),
            mode=lax.GatherScatterMode.PROMISE_IN_BOUNDS)

    def _allmax(v):
        for d in (8, 4, 2, 1):
            v = jnp.maximum(v, _shuffle(v, jnp.bitwise_xor(lane, d)))
        return v

    def _allmin(v):
        for d in (8, 4, 2, 1):
            v = jnp.minimum(v, _shuffle(v, jnp.bitwise_xor(lane, d)))
        return v

    def _allsum(v):
        for d in (8, 4, 2, 1):
            v = v + _shuffle(v, jnp.bitwise_xor(lane, d))
        return v

    def _store1(ref, ivec, vvec):
        plsc.store_scatter(ref, [ivec], vvec, mask=lane0)

    def process_pair(rowa, rowb, sa, sb):
        def chunk_body(c, carry2):
            base = c * CHUNK
            acca = jnp.full((L,), NEGINF, jnp.float32)
            accb = jnp.full((L,), NEGINF, jnp.float32)
            for j in range(CHUNK // L):
                off = pl.ds(base + j * L, L)
                acca = jnp.maximum(acca, sa[off])
                accb = jnp.maximum(accb, sb[off])
            cl = jnp.bitwise_and(c, L - 1)
            hit = lane == cl
            ma = carry2[:NG]
            mb = carry2[NG:]
            g0 = lax.shift_right_logical(c, 4)
            va_ = _allmax(acca)
            vb_ = _allmax(accb)
            ma = tuple(
                jnp.where(jnp.logical_and(g0 == g, hit), va_, ma[g])
                for g in range(NG))
            mb = tuple(
                jnp.where(jnp.logical_and(g0 == g, hit), vb_, mb[g])
                for g in range(NG))
            return ma + mb

        init = tuple(jnp.full((L,), NEGINF, jnp.float32) for _ in range(2 * NG))
        maxima = lax.fori_loop(0, NCH, chunk_body, init)

        def one_pick(sbuf, ovals, oidx, k, m1r):
            bv = m1r[0]
            bi = lane
            for g in range(1, NG):
                gt = m1r[g] > bv
                bv = jnp.where(gt, m1r[g], bv)
                bi = jnp.where(gt, lane + g * L, bi)
            m = _allmax(bv)
            cstar = _allmin(jnp.where(bv == m, bi, BIG))
            base = cstar * CHUNK
            cand = jnp.full((L,), BIG, jnp.int32)
            acc2 = jnp.full((L,), NEGINF, jnp.float32)
            occ = jnp.zeros((L,), jnp.int32)
            for j in range(CHUNK // L):
                pos = base + j * L + lane
                s = plsc.load_gather(sbuf, [pos])
                ism = s == m
                cand = jnp.minimum(cand, jnp.where(ism, pos, BIG))
                acc2 = jnp.maximum(acc2, jnp.where(ism, NEGINF, s))
                occ = occ + jnp.where(ism, 1, 0)
            idx = _allmin(cand)
            kvec = jnp.full((L,), k, jnp.int32)
            _store1(ovals, kvec, m)
            _store1(oidx, kvec, idx)
            _store1(sbuf, idx, jnp.full((L,), NEGINF, jnp.float32))
            newm = jnp.where(_allsum(occ) > 1, m, _allmax(acc2))
            l0 = jnp.bitwise_and(cstar, L - 1)
            g0 = lax.shift_right_logical(cstar, 4)
            hit = lane == l0
            return tuple(
                jnp.where(jnp.logical_and(g0 == g, hit), newm, m1r[g])
                for g in range(NG))

        def pick_body(k, carry2):
            m1ra = carry2[:NG]
            m1rb = carry2[NG:]
            m1ra = one_pick(sa, va, ia, k, m1ra)
            m1rb = one_pick(sb, vb, ib, k, m1rb)
            return m1ra + m1rb

        lax.fori_loop(0, K, pick_body, maxima)
        pltpu.sync_copy(va, vals_hbm.at[rowa])
        pltpu.sync_copy(ia, idx_hbm.at[rowa])
        pltpu.sync_copy(vb, vals_hbm.at[rowb])
        pltpu.sync_copy(ib, idx_hbm.at[rowb])

    base_row = wid * RPW
    h0a = pltpu.async_copy(x_hbm.at[base_row + 0], bufa, sem0)
    h0b = pltpu.async_copy(x_hbm.at[base_row + 1], bufb, sem0)
    h1a = pltpu.async_copy(x_hbm.at[base_row + 2], bufc, sem1)
    h0a.wait()
    h0b.wait()
    process_pair(base_row + 0, base_row + 1, bufa, bufb)
    h1b = pltpu.async_copy(x_hbm.at[base_row + 3], bufb, sem1)
    h1a.wait()
    h1b.wait()
    process_pair(base_row + 2, base_row + 3, bufc, bufb)


def kernel(x):
    vals, idx = _topk_sc(x)
    return vals, idx
